# Initial kernel scaffold; baseline (speedup 1.0000x reference)
import jax
import jax.numpy as jnp
from jax.experimental import pallas as pl


def kernel(x, edge_index, params):
    raise NotImplementedError("write your pallas kernel here")


# speedup vs baseline: 15.3665x; 15.3665x over previous
import functools

import jax
import jax.numpy as jnp
from jax import lax
from jax.experimental import pallas as pl
from jax.experimental.pallas import tpu as pltpu
from jax.experimental.pallas import tpu_sc as plsc

N_NODES = 10000
NPAD = 10240
ROWS_PER_TILE = NPAD // 16
D = 128
EB = 64
NTILES = 32
BLOCKS_PER_TILE = 84
NREFILL = 3
BPR = BLOCKS_PER_TILE // NREFILL
EPAD = NTILES * BLOCKS_PER_TILE * EB
BM = 256


def _make_edge_kernel(heads):
    mesh = plsc.VectorSubcoreMesh(core_axis_name="c", subcore_axis_name="s")

    @functools.partial(
        pl.kernel,
        mesh=mesh,
        out_type=(
            jax.ShapeDtypeStruct((2, NPAD, D), jnp.float32),
            jax.ShapeDtypeStruct((NTILES, BLOCKS_PER_TILE * EB * 16),
                                 jnp.float32),
        ),
        scratch_types=[
            pltpu.VMEM((2 * BPR, EB), jnp.int32),
            pltpu.VMEM((EB, D), jnp.float32),
            pltpu.VMEM((EB, D), jnp.float32),
            pltpu.VMEM((EB * 16,), jnp.float32),
            pltpu.VMEM((8, 16), jnp.float32),
            pltpu.VMEM_SHARED((NPAD, D), jnp.float32),
            pltpu.SemaphoreType.DMA,
            pltpu.SemaphoreType.DMA,
        ],
        compiler_params=pltpu.CompilerParams(needs_layout_passes=False),
    )
    def edge_kernel(xl_hbm, xr_hbm, idx_hbm, att_hbm,
                    out_hbm, p_hbm,
                    idx_v, xl_v, xr_v, p_v, att_v,
                    acc_s, sem1, sem2):
        cid = lax.axis_index("c")
        sid = lax.axis_index("s")
        wid = sid * 2 + cid

        pltpu.sync_copy(att_hbm, att_v)

        zero16 = jnp.zeros((16,), jnp.float32)
        lane = lax.iota(jnp.int32, 16)

        def zero_row(i, carry):
            for j in range(8):
                xl_v[i, pl.ds(16 * j, 16)] = zero16
            p_v[pl.ds(16 * i, 16)] = zero16
            return carry
        lax.fori_loop(0, EB, zero_row, 0)

        base = sid * ROWS_PER_TILE
        nfull, rem = divmod(ROWS_PER_TILE, EB)
        for k in range(nfull):
            pltpu.sync_copy(xl_v, acc_s.at[pl.ds(base + k * EB, EB)])
        plsc.subcore_barrier()

        def refill_body(r, carry):
            pltpu.sync_copy(idx_hbm.at[wid, r], idx_v)

            def clamp_body(row, c0):
                for k4 in range(EB // 16):
                    v = idx_v[row, pl.ds(16 * k4, 16)]
                    idx_v[row, pl.ds(16 * k4, 16)] = jnp.clip(v, 0, NPAD - 1)
                return c0
            lax.fori_loop(0, 2 * BPR, clamp_body, 0)

            def block_body(j, c1):
                srow = idx_v.at[2 * j]
                drow = idx_v.at[2 * j + 1]
                cp1 = pltpu.async_copy(xl_hbm.at[srow], xl_v, sem1)
                cp2 = pltpu.async_copy(xr_hbm.at[drow], xr_v, sem2)
                cp1.wait()
                cp2.wait()

                def group_body(g, c2):
                    evec = lane + g * 16
                    if heads == 8:
                        for h in range(8):
                            attv = att_v[h]
                            sacc = zero16
                            for ci in range(16):
                                c = 16 * h + ci
                                cvec = jnp.full((16,), c, jnp.int32)
                                xlv = plsc.load_gather(xl_v, [evec, cvec])
                                xrv = plsc.load_gather(xr_v, [evec, cvec])
                                t = xlv + xrv
                                t = jnp.maximum(t, 0.2 * t)
                                sacc = sacc + t * attv[ci]
                            hvec = jnp.full((16,), h, jnp.int32)
                            plsc.store_scatter(p_v, [evec * 16 + hvec],
                                               jnp.exp(sacc))
                    else:
                        sacc = zero16
                        for hb in range(8):
                            attv = att_v[hb]
                            for ci in range(16):
                                c = 16 * hb + ci
                                cvec = jnp.full((16,), c, jnp.int32)
                                xlv = plsc.load_gather(xl_v, [evec, cvec])
                                xrv = plsc.load_gather(xr_v, [evec, cvec])
                                t = xlv + xrv
                                t = jnp.maximum(t, 0.2 * t)
                                sacc = sacc + t * attv[ci]
                        plsc.store_scatter(p_v, [evec * 16], jnp.exp(sacc))
                    return c2
                lax.fori_loop(0, EB // 16, group_body, 0)

                def wrow_body(e, c2):
                    pv = p_v[pl.ds(e * 16, 16)]
                    for h in range(8):
                        ph = pv[h] if heads == 8 else pv[0]
                        xl_v[e, pl.ds(16 * h, 16)] = (
                            xl_v[e, pl.ds(16 * h, 16)] * ph)
                    return c2
                lax.fori_loop(0, EB, wrow_body, 0)

                pltpu.sync_copy(xl_v, acc_s.at[drow], add=True)
                bg = r * BPR + j
                pltpu.sync_copy(
                    p_v, p_hbm.at[wid, pl.ds(bg * (EB * 16), EB * 16)])
                return c1
            lax.fori_loop(0, BPR, block_body, 0)
            return carry
        lax.fori_loop(0, NREFILL, refill_body, 0)

        plsc.subcore_barrier()
        for k in range(nfull):
            r0 = base + k * EB
            pltpu.sync_copy(acc_s.at[pl.ds(r0, EB)], out_hbm.at[cid, pl.ds(r0, EB)])

    return edge_kernel


_edge8 = _make_edge_kernel(8)
_edge1 = _make_edge_kernel(1)


def _make_den_kernel(heads):
    mesh = plsc.VectorSubcoreMesh(core_axis_name="c", subcore_axis_name="s")

    @functools.partial(
        pl.kernel,
        mesh=mesh,
        out_type=jax.ShapeDtypeStruct((2, NPAD, D), jnp.float32),
        scratch_types=[
            pltpu.VMEM((2 * BPR, EB), jnp.int32),
            pltpu.VMEM((EB * 16,), jnp.float32),
            pltpu.VMEM((EB, D), jnp.float32),
            pltpu.VMEM_SHARED((NPAD, D), jnp.float32),
        ],
        compiler_params=pltpu.CompilerParams(needs_layout_passes=False),
    )
    def den_kernel(idx_hbm, p_hbm, den_hbm, idx_v, p_v, dx_v, den_s):
        cid = lax.axis_index("c")
        sid = lax.axis_index("s")
        wid = sid * 2 + cid
        zero16 = jnp.zeros((16,), jnp.float32)

        def zero_row(i, carry):
            for j in range(8):
                dx_v[i, pl.ds(16 * j, 16)] = zero16
            return carry
        lax.fori_loop(0, EB, zero_row, 0)

        base = sid * ROWS_PER_TILE
        nfull = ROWS_PER_TILE // EB
        for k in range(nfull):
            pltpu.sync_copy(dx_v, den_s.at[pl.ds(base + k * EB, EB)])
        plsc.subcore_barrier()

        def refill_body(r, carry):
            pltpu.sync_copy(idx_hbm.at[wid, r], idx_v)

            def block_body(j, c1):
                drow = idx_v.at[2 * j + 1]
                bg = r * BPR + j
                pltpu.sync_copy(
                    p_hbm.at[wid, pl.ds(bg * (EB * 16), EB * 16)], p_v)

                def edge_body(e, c2):
                    pv = p_v[pl.ds(e * 16, 16)]
                    for h in range(8):
                        ph = pv[h] if heads == 8 else pv[0]
                        dx_v[e, pl.ds(16 * h, 16)] = zero16 + ph
                    return c2
                lax.fori_loop(0, EB, edge_body, 0)

                pltpu.sync_copy(dx_v, den_s.at[drow], add=True)
                return c1
            lax.fori_loop(0, BPR, block_body, 0)
            return carry
        lax.fori_loop(0, NREFILL, refill_body, 0)

        plsc.subcore_barrier()
        for k in range(nfull):
            r0 = base + k * EB
            pltpu.sync_copy(den_s.at[pl.ds(r0, EB)],
                            den_hbm.at[cid, pl.ds(r0, EB)])

    return den_kernel


_den8 = _make_den_kernel(8)
_den1 = _make_den_kernel(1)


def _proj_body(h_ref, w_ref, b_ref, xl_ref, xr_ref):
    y = jnp.dot(h_ref[...], w_ref[...],
                preferred_element_type=jnp.float32,
                precision=lax.Precision.HIGHEST)
    y = y + b_ref[...]
    xl_ref[...] = y[:, :D]
    xr_ref[...] = y[:, D:]


def _proj(h_pad, w_cat, b_cat):
    return pl.pallas_call(
        _proj_body,
        grid=(NPAD // BM,),
        in_specs=[
            pl.BlockSpec((BM, D), lambda i: (i, 0)),
            pl.BlockSpec((D, 2 * D), lambda i: (0, 0)),
            pl.BlockSpec((1, 2 * D), lambda i: (0, 0)),
        ],
        out_specs=[
            pl.BlockSpec((BM, D), lambda i: (i, 0)),
            pl.BlockSpec((BM, D), lambda i: (i, 0)),
        ],
        out_shape=[jax.ShapeDtypeStruct((NPAD, D), jnp.float32)] * 2,
    )(h_pad, w_cat, b_cat)


def _expand_mat():
    return jnp.kron(jnp.eye(8, dtype=jnp.float32),
                    jnp.ones((1, 16), jnp.float32))


def _combine_body(a_ref, d_ref, bias_ref, g_ref, b_ref, o_ref):
    a = a_ref[0] + a_ref[1]
    dsum = d_ref[0] + d_ref[1]
    u = a / (dsum + 1e-16) + bias_ref[...]
    mu = jnp.mean(u, axis=-1, keepdims=True)
    var = jnp.mean((u - mu) ** 2, axis=-1, keepdims=True)
    hn = (u - mu) / jnp.sqrt(var + 1e-5) * g_ref[...] + b_ref[...]
    o_ref[...] = hn * 0.5 * (1.0 + lax.erf(hn * 0.7071067811865476))


def _combine_ln_gelu(acc, den, bias, g, b):
    return pl.pallas_call(
        _combine_body,
        grid=(NPAD // BM,),
        in_specs=[
            pl.BlockSpec((2, BM, D), lambda i: (0, i, 0)),
            pl.BlockSpec((2, BM, D), lambda i: (0, i, 0)),
            pl.BlockSpec((1, D), lambda i: (0, 0)),
            pl.BlockSpec((1, D), lambda i: (0, 0)),
            pl.BlockSpec((1, D), lambda i: (0, 0)),
        ],
        out_specs=pl.BlockSpec((BM, D), lambda i: (i, 0)),
        out_shape=jax.ShapeDtypeStruct((NPAD, D), jnp.float32),
    )(acc, den, bias, g, b)


def _final_body(a_ref, d_ref, bias_ref, o_ref):
    a = a_ref[0] + a_ref[1]
    dsum = d_ref[0] + d_ref[1]
    o_ref[...] = a / (dsum + 1e-16) + bias_ref[...]


def _final_combine(acc, den, bias):
    return pl.pallas_call(
        _final_body,
        grid=(NPAD // BM,),
        in_specs=[
            pl.BlockSpec((2, BM, D), lambda i: (0, i, 0)),
            pl.BlockSpec((2, BM, D), lambda i: (0, i, 0)),
            pl.BlockSpec((1, D), lambda i: (0, 0)),
        ],
        out_specs=pl.BlockSpec((BM, D), lambda i: (i, 0)),
        out_shape=jax.ShapeDtypeStruct((NPAD, D), jnp.float32),
    )(acc, den, bias)


def kernel(x, edge_index, params):
    convs = params['convs']
    norms = params['norms']
    n = N_NODES

    loop = jnp.arange(n, dtype=edge_index.dtype)
    src = jnp.concatenate([edge_index[0], loop])
    dst = jnp.concatenate([edge_index[1], loop])
    pad = EPAD - src.shape[0]
    fill = jnp.full((pad,), n, dtype=src.dtype)
    src3 = jnp.concatenate([src, fill]).reshape(NTILES, BLOCKS_PER_TILE, EB)
    dst3 = jnp.concatenate([dst, fill]).reshape(NTILES, BLOCKS_PER_TILE, EB)
    idx4 = jnp.stack([src3, dst3], axis=2).reshape(
        NTILES, NREFILL, 2 * BPR, EB).astype(jnp.int32)

    h = jnp.pad(x, ((0, NPAD - n), (0, 0)))
    for i in range(3):
        p = convs[i]
        w_cat = jnp.concatenate([p['W_l'], p['W_r']], axis=1)
        b_cat = jnp.concatenate([p['b_l'], p['b_r']])[None, :]
        xl, xr = _proj(h, w_cat, b_cat)
        att = p['att'].reshape(8, 16)
        acc, pexp = _edge8(xl, xr, idx4, att)
        den = _den8(idx4, pexp)
        h = _combine_ln_gelu(acc, den, p['bias'][None, :],
                             norms[i]['g'][None, :], norms[i]['b'][None, :])

    p = convs[3]
    w_cat = jnp.concatenate([p['W_l'], p['W_r']], axis=1)
    b_cat = jnp.concatenate([p['b_l'], p['b_r']])[None, :]
    xl, xr = _proj(h, w_cat, b_cat)
    att = p['att'].reshape(8, 16)
    acc, pexp = _edge1(xl, xr, idx4, att)
    den = _den1(idx4, pexp)
    out = _final_combine(acc, den, p['bias'][None, :])
    return out[:n]
